# Initial kernel scaffold; baseline (speedup 1.0000x reference)
import jax
import jax.numpy as jnp
from jax.experimental import pallas as pl


def kernel(w1, b1, s2, w2, b2, s3, w3, b3, fc1_w, fc1_b, fc2_w, fc2_b, x):
    raise NotImplementedError("write your pallas kernel here")


# speedup vs baseline: 1.6222x; 1.6222x over previous
import jax
import jax.numpy as jnp
from jax.experimental import pallas as pl
from jax.experimental.pallas import tpu as pltpu

_LANE = 128
_NB = 8


def _im2col_T(x, k=8, stride=4):
    n, c, h, w = x.shape
    oh = (h - k) // stride + 1
    ow = (w - k) // stride + 1
    rows = (jnp.arange(oh) * stride)[:, None] + jnp.arange(k)[None, :]
    cols = (jnp.arange(ow) * stride)[:, None] + jnp.arange(k)[None, :]
    p = x[:, :, rows[:, :, None, None], cols[None, None, :, :]]
    p = p.transpose(0, 1, 3, 5, 2, 4)
    return p.reshape(n, c * k * k, oh * ow)


def _conv_stack_kernel(p1_ref, w1_ref, b1_ref, s2_ref, w2_ref, b2_ref,
                       s3_ref, w3_ref, b3_ref, o_ref, h1s, z2, h2s, z3):
    f32 = jnp.float32
    bf16 = jnp.bfloat16
    nb = o_ref.shape[0]

    for i in range(nb):
        h1 = jnp.dot(w1_ref[...], p1_ref[i], preferred_element_type=f32)
        h1s[i * 32:(i + 1) * 32, :] = jnp.maximum(h1 + b1_ref[...], 0.0).astype(bf16)

    g2 = jnp.dot(h1s[...], s2_ref[...], preferred_element_type=f32).astype(bf16)
    t2 = s2_ref.shape[1] // _LANE
    cin2 = w2_ref.shape[1] // t2
    for i in range(nb):
        for t in range(t2):
            z2[t * cin2:(t + 1) * cin2, i * _LANE:(i + 1) * _LANE] = \
                g2[i * cin2:(i + 1) * cin2, t * _LANE:(t + 1) * _LANE]
    h2 = jnp.dot(w2_ref[...], z2[...], preferred_element_type=f32)
    h2 = jnp.maximum(h2 + b2_ref[...], 0.0).astype(bf16)

    for i in range(nb):
        h2s[i * 64:(i + 1) * 64, :] = h2[:, i * _LANE:(i + 1) * _LANE]
    g3 = jnp.dot(h2s[...], s3_ref[...], preferred_element_type=f32).astype(bf16)
    t3 = s3_ref.shape[1] // _LANE
    cin3 = w3_ref.shape[1] // t3
    for i in range(nb):
        for t in range(t3):
            z3[t * cin3:(t + 1) * cin3, i * _LANE:(i + 1) * _LANE] = \
                g3[i * cin3:(i + 1) * cin3, t * _LANE:(t + 1) * _LANE]
    h3 = jnp.dot(w3_ref[...], z3[...], preferred_element_type=f32)
    h3 = jnp.maximum(h3 + b3_ref[...], 0.0)

    m3 = o_ref.shape[2]
    for i in range(nb):
        o_ref[i] = h3[:, i * _LANE:i * _LANE + m3].astype(o_ref.dtype)


def _conv_stack(p1, w1, b1, s2, w2, b2, s3, w3, b3, nb):
    n, k1, m1 = p1.shape
    c3 = w3.shape[0]
    hw1 = int(round(m1 ** 0.5))
    hw2 = (hw1 - 4) // 2 + 1
    hw3 = hw2 - 2
    m3 = hw3 * hw3
    return pl.pallas_call(
        _conv_stack_kernel,
        out_shape=jax.ShapeDtypeStruct((n, c3, m3), jnp.bfloat16),
        grid=(n // nb,),
        in_specs=[
            pl.BlockSpec((nb, k1, m1), lambda i: (i, 0, 0)),
            pl.BlockSpec(w1.shape, lambda i: (0, 0)),
            pl.BlockSpec(b1.shape, lambda i: (0, 0)),
            pl.BlockSpec(s2.shape, lambda i: (0, 0)),
            pl.BlockSpec(w2.shape, lambda i: (0, 0)),
            pl.BlockSpec(b2.shape, lambda i: (0, 0)),
            pl.BlockSpec(s3.shape, lambda i: (0, 0)),
            pl.BlockSpec(w3.shape, lambda i: (0, 0)),
            pl.BlockSpec(b3.shape, lambda i: (0, 0)),
        ],
        out_specs=pl.BlockSpec((nb, c3, m3), lambda i: (i, 0, 0)),
        scratch_shapes=[
            pltpu.VMEM((nb * 32, m1), jnp.bfloat16),
            pltpu.VMEM((w2.shape[1], nb * _LANE), jnp.bfloat16),
            pltpu.VMEM((nb * 64, _LANE), jnp.bfloat16),
            pltpu.VMEM((w3.shape[1], nb * _LANE), jnp.bfloat16),
        ],
        compiler_params=pltpu.CompilerParams(
            dimension_semantics=("parallel",),
            vmem_limit_bytes=64 << 20,
        ),
    )(p1, w1, b1, s2, w2, b2, s3, w3, b3)


def _fc_kernel(x_ref, w1_ref, b1_ref, w2_ref, b2_ref, o_ref):
    h = jnp.dot(x_ref[...], w1_ref[...], preferred_element_type=jnp.float32)
    h = jnp.maximum(h + b1_ref[...], 0.0).astype(jnp.bfloat16)
    y = jnp.dot(h, w2_ref[...], preferred_element_type=jnp.float32) + b2_ref[...]
    o_ref[...] = y.astype(o_ref.dtype)


def _fc(x, w1, b1, w2, b2):
    m, k = x.shape
    n1 = w1.shape[1]
    n2 = w2.shape[1]
    gm = 2 if m % 2 == 0 else 1
    return pl.pallas_call(
        _fc_kernel,
        out_shape=jax.ShapeDtypeStruct((m, n2), jnp.float32),
        grid=(gm,),
        in_specs=[
            pl.BlockSpec((m // gm, k), lambda i: (i, 0)),
            pl.BlockSpec((k, n1), lambda i: (0, 0)),
            pl.BlockSpec((1, n1), lambda i: (0, 0)),
            pl.BlockSpec((n1, n2), lambda i: (0, 0)),
            pl.BlockSpec((1, n2), lambda i: (0, 0)),
        ],
        out_specs=pl.BlockSpec((m // gm, n2), lambda i: (i, 0)),
        compiler_params=pltpu.CompilerParams(
            dimension_semantics=("parallel",),
            vmem_limit_bytes=64 << 20,
        ),
    )(x, w1, b1, w2, b2)


def kernel(w1, b1, s2, w2, b2, s3, w3, b3, fc1_w, fc1_b, fc2_w, fc2_b, x):
    bf16 = jnp.bfloat16
    n = x.shape[0]
    nb = next(v for v in (_NB, 4, 2, 1) if n % v == 0)
    p1 = _im2col_T(x.astype(bf16))
    h = _conv_stack(p1, w1.astype(bf16), b1, s2.astype(bf16), w2.astype(bf16),
                    b2, s3.astype(bf16), w3.astype(bf16), b3, nb)
    h = h.reshape(n, -1)
    return _fc(h, fc1_w.astype(bf16), fc1_b, fc2_w.astype(bf16), fc2_b)
